# baseline (speedup 1.0000x reference)
import functools

import jax
import jax.numpy as jnp
from jax.experimental import pallas as pl
from jax.experimental.pallas import tpu as pltpu

N_IN = 4096
N_HID = 2048
K_SP = 204
B = 4096


def _encode_body(x_ref, w_ref, b_ref, o_ref):
    k = pl.program_id(2)
    nk = pl.num_programs(2)

    @pl.when(k == 0)
    def _():
        o_ref[...] = jnp.zeros_like(o_ref)

    o_ref[...] += jax.lax.dot_general(
        x_ref[...].astype(jnp.bfloat16), w_ref[...].astype(jnp.bfloat16),
        (((1,), (1,)), ((), ())),
        preferred_element_type=jnp.float32,
    )

    @pl.when(k == nk - 1)
    def _():
        o_ref[...] = jax.nn.sigmoid(o_ref[...] + b_ref[...])


def _encode(x, w_enc, b_enc):
    bm, bn, bk = 512, 512, 1024
    grid = (B // bm, N_HID // bn, N_IN // bk)
    return pl.pallas_call(
        _encode_body,
        grid=grid,
        in_specs=[
            pl.BlockSpec((bm, bk), lambda i, j, k: (i, k)),
            pl.BlockSpec((bn, bk), lambda i, j, k: (j, k)),
            pl.BlockSpec((1, bn), lambda i, j, k: (0, j)),
        ],
        out_specs=pl.BlockSpec((bm, bn), lambda i, j, k: (i, j)),
        out_shape=jax.ShapeDtypeStruct((B, N_HID), jnp.float32),
    )(x, w_enc, b_enc.reshape(1, N_HID))


def _mask_body(enc_ref, mask_ref):
    i = pl.program_id(0)
    bits = jax.lax.bitcast_convert_type(enc_ref[...], jnp.int32)

    lo = jnp.zeros((bits.shape[0], 1), jnp.int32)
    hi = jnp.full((bits.shape[0], 1), 0x3F800000, jnp.int32)

    def step(_, carry):
        lo, hi = carry
        mid = (lo + hi) >> 1
        cnt = jnp.sum((bits <= mid).astype(jnp.int32), axis=1, keepdims=True)
        ge = cnt >= K_SP
        return jnp.where(ge, lo, mid + 1), jnp.where(ge, mid, hi)

    lo, hi = jax.lax.fori_loop(0, 30, step, (lo, hi))
    sel = (bits <= lo).astype(jnp.float32)
    part = jnp.max(sel, axis=0, keepdims=True)

    @pl.when(i == 0)
    def _():
        mask_ref[...] = jnp.zeros_like(mask_ref)

    mask_ref[...] = jnp.maximum(mask_ref[...], part)


def _mask(encoded):
    bm = 256
    return pl.pallas_call(
        _mask_body,
        grid=(B // bm,),
        in_specs=[pl.BlockSpec((bm, N_HID), lambda i: (i, 0))],
        out_specs=pl.BlockSpec((1, N_HID), lambda i: (0, 0)),
        out_shape=jax.ShapeDtypeStruct((1, N_HID), jnp.float32),
    )(encoded)


def _decode_body(enc_ref, m_ref, w_ref, b_ref, o_ref):
    k = pl.program_id(2)
    nk = pl.num_programs(2)

    @pl.when(k == 0)
    def _():
        o_ref[...] = jnp.zeros_like(o_ref)

    e = (enc_ref[...] * m_ref[...]).astype(jnp.bfloat16)
    o_ref[...] += jax.lax.dot_general(
        e, w_ref[...].astype(jnp.bfloat16), (((1,), (1,)), ((), ())),
        preferred_element_type=jnp.float32,
    )

    @pl.when(k == nk - 1)
    def _():
        o_ref[...] += b_ref[...]


def _decode(encoded, mask, w_dec, b_dec):
    bm, bn, bk = 512, 512, 1024
    grid = (B // bm, N_IN // bn, N_HID // bk)
    return pl.pallas_call(
        _decode_body,
        grid=grid,
        in_specs=[
            pl.BlockSpec((bm, bk), lambda i, j, k: (i, k)),
            pl.BlockSpec((1, bk), lambda i, j, k: (0, k)),
            pl.BlockSpec((bn, bk), lambda i, j, k: (j, k)),
            pl.BlockSpec((1, bn), lambda i, j, k: (0, j)),
        ],
        out_specs=pl.BlockSpec((bm, bn), lambda i, j, k: (i, j)),
        out_shape=jax.ShapeDtypeStruct((B, N_IN), jnp.float32),
    )(encoded, mask, w_dec, b_dec.reshape(1, N_IN))


def kernel(input, W_enc, b_enc, W_dec, b_dec):
    encoded = _encode(input, W_enc, b_enc)
    mask = _mask(encoded)
    return _decode(encoded, mask, W_dec, b_dec)


# speedup vs baseline: 1.5486x; 1.5486x over previous
import jax
import jax.numpy as jnp
from jax.experimental import pallas as pl
from jax.experimental.pallas import tpu as pltpu

N_IN = 4096
N_HID = 2048
K_SP = 204
B = 4096


def _encode_body(x_ref, w_ref, b_ref, o_ref):
    acc = jax.lax.dot_general(
        x_ref[...], w_ref[...], (((1,), (1,)), ((), ())),
        preferred_element_type=jnp.float32,
    )
    o_ref[...] = jax.nn.sigmoid(acc + b_ref[...])


def _encode(x_bf, w_enc_bf, b_enc):
    bm, bn = 1024, N_HID
    grid = (B // bm,)
    return pl.pallas_call(
        _encode_body,
        grid=grid,
        in_specs=[
            pl.BlockSpec((bm, N_IN), lambda i: (i, 0)),
            pl.BlockSpec((bn, N_IN), lambda i: (0, 0)),
            pl.BlockSpec((1, bn), lambda i: (0, 0)),
        ],
        out_specs=pl.BlockSpec((bm, bn), lambda i: (i, 0)),
        out_shape=jax.ShapeDtypeStruct((B, N_HID), jnp.float32),
    )(x_bf, w_enc_bf, b_enc.reshape(1, N_HID))


def _mask_body(enc_ref, mask_ref):
    i = pl.program_id(0)
    bits = jax.lax.bitcast_convert_type(enc_ref[...], jnp.int32)

    lo = jnp.zeros((bits.shape[0], 1), jnp.int32)
    hi = jnp.full((bits.shape[0], 1), 0x3F800000, jnp.int32)

    def step(_, carry):
        lo, hi = carry
        mid = (lo + hi) >> 1
        cnt = jnp.sum((bits <= mid).astype(jnp.int32), axis=1, keepdims=True)
        ge = cnt >= K_SP
        return jnp.where(ge, lo, mid + 1), jnp.where(ge, mid, hi)

    lo, hi = jax.lax.fori_loop(0, 30, step, (lo, hi))
    sel = (bits <= lo).astype(jnp.float32)
    part = jnp.max(sel, axis=0, keepdims=True)

    @pl.when(i == 0)
    def _():
        mask_ref[...] = jnp.zeros_like(mask_ref)

    mask_ref[...] = jnp.maximum(mask_ref[...], part)


def _mask(encoded):
    bm = 256
    return pl.pallas_call(
        _mask_body,
        grid=(B // bm,),
        in_specs=[pl.BlockSpec((bm, N_HID), lambda i: (i, 0))],
        out_specs=pl.BlockSpec((1, N_HID), lambda i: (0, 0)),
        out_shape=jax.ShapeDtypeStruct((1, N_HID), jnp.float32),
    )(encoded)


def _decode_body(enc_ref, m_ref, w_ref, b_ref, o_ref):
    e = (enc_ref[...] * m_ref[...]).astype(jnp.bfloat16)
    acc = jax.lax.dot_general(
        e, w_ref[...], (((1,), (1,)), ((), ())),
        preferred_element_type=jnp.float32,
    )
    o_ref[...] = acc + b_ref[...]


def _decode(encoded, mask, w_dec_bf, b_dec):
    bm, bn = 1024, 2048
    grid = (B // bm, N_IN // bn)
    return pl.pallas_call(
        _decode_body,
        grid=grid,
        in_specs=[
            pl.BlockSpec((bm, N_HID), lambda i, j: (i, 0)),
            pl.BlockSpec((1, N_HID), lambda i, j: (0, 0)),
            pl.BlockSpec((bn, N_HID), lambda i, j: (j, 0)),
            pl.BlockSpec((1, bn), lambda i, j: (0, j)),
        ],
        out_specs=pl.BlockSpec((bm, bn), lambda i, j: (i, j)),
        out_shape=jax.ShapeDtypeStruct((B, N_IN), jnp.float32),
    )(encoded, mask, w_dec_bf, b_dec.reshape(1, N_IN))


def kernel(input, W_enc, b_enc, W_dec, b_dec):
    x_bf = input.astype(jnp.bfloat16)
    w_enc_bf = W_enc.astype(jnp.bfloat16)
    w_dec_bf = W_dec.astype(jnp.bfloat16)
    encoded = _encode(x_bf, w_enc_bf, b_enc)
    mask = _mask(encoded)
    return _decode(encoded, mask, w_dec_bf, b_dec)
